# Initial kernel scaffold; baseline (speedup 1.0000x reference)
import jax
import jax.numpy as jnp
from jax.experimental import pallas as pl


def kernel(x, edge_index, W, b):
    raise NotImplementedError("write your pallas kernel here")


# speedup vs baseline: 5.3660x; 5.3660x over previous
import functools

import jax
import jax.numpy as jnp
from jax import lax
from jax.experimental import pallas as pl
from jax.experimental.pallas import tpu as pltpu
from jax.experimental.pallas import tpu_sc as plsc

N_NODES = 10000
D_FEAT = 128
N_EDGES = 320000

NC = 2
NS = 16
N_TILES = NC * NS

EDGES_PER_TILE = N_EDGES // N_TILES
CHUNK = 80
ITERS = EDGES_PER_TILE // CHUNK
N_PAD = 10240
ROWS_PER_TILE = N_PAD // NS


def _sc_scatter_add(x, src, dst, zeros):
  mesh = plsc.VectorSubcoreMesh(core_axis_name="c", subcore_axis_name="s")

  @functools.partial(
      pl.kernel,
      mesh=mesh,
      out_type=jax.ShapeDtypeStruct((NC, N_PAD, D_FEAT), jnp.float32),
      scratch_types=[
          pltpu.VMEM((CHUNK,), jnp.int32),
          pltpu.VMEM((CHUNK,), jnp.int32),
          pltpu.VMEM((CHUNK, D_FEAT), jnp.float32),
          pltpu.VMEM_SHARED((N_PAD, D_FEAT), jnp.float32),
          pltpu.SemaphoreType.DMA,
      ],
  )
  def body(x_hbm, src_hbm, dst_hbm, zeros_hbm, out_hbm,
           sidx, didx, rows, ysh, sem):
    c = lax.axis_index("c")
    s = lax.axis_index("s")
    row0 = pl.multiple_of(s * ROWS_PER_TILE, 8)
    pltpu.sync_copy(zeros_hbm.at[pl.ds(row0, ROWS_PER_TILE)],
                    ysh.at[pl.ds(row0, ROWS_PER_TILE)])
    plsc.subcore_barrier()

    base = (c * NS + s) * EDGES_PER_TILE

    def step(i, carry):
      off = pl.multiple_of(base + i * CHUNK, 8)
      pltpu.sync_copy(src_hbm.at[pl.ds(off, CHUNK)], sidx)
      pltpu.sync_copy(dst_hbm.at[pl.ds(off, CHUNK)], didx)
      pltpu.async_copy(x_hbm.at[sidx], rows, sem).wait()
      pltpu.sync_copy(rows, ysh.at[didx], add=True)
      return carry

    lax.fori_loop(0, ITERS, step, 0)
    plsc.subcore_barrier()
    pltpu.sync_copy(ysh.at[pl.ds(row0, ROWS_PER_TILE)],
                    out_hbm.at[c].at[pl.ds(row0, ROWS_PER_TILE)])

  return body(x, src, dst, zeros)


def _tc_linear(y2, W, b):
  BM = 400

  def body(y_ref, w_ref, b_ref, o_ref):
    ysum = y_ref[0] + y_ref[1]
    o_ref[...] = lax.dot_general(
        ysum, w_ref[...], (((1,), (1,)), ((), ())),
        preferred_element_type=jnp.float32) + b_ref[...]

  return pl.pallas_call(
      body,
      grid=(N_NODES // BM,),
      in_specs=[
          pl.BlockSpec((NC, BM, D_FEAT), lambda i: (0, i, 0)),
          pl.BlockSpec((D_FEAT, D_FEAT), lambda i: (0, 0)),
          pl.BlockSpec((1, D_FEAT), lambda i: (0, 0)),
      ],
      out_specs=pl.BlockSpec((BM, D_FEAT), lambda i: (i, 0)),
      out_shape=jax.ShapeDtypeStruct((N_NODES, D_FEAT), jnp.float32),
  )(y2, W, b.reshape(1, D_FEAT))


def kernel(x, edge_index, W, b):
  src = edge_index[0].astype(jnp.int32)
  dst = edge_index[1].astype(jnp.int32)
  zeros = jnp.zeros((N_PAD, D_FEAT), jnp.float32)
  y2 = _sc_scatter_add(x, src, dst, zeros)
  return _tc_linear(y2, W, b)


# speedup vs baseline: 11.2667x; 2.0996x over previous
import functools

import jax
import jax.numpy as jnp
from jax import lax
from jax.experimental import pallas as pl
from jax.experimental.pallas import tpu as pltpu
from jax.experimental.pallas import tpu_sc as plsc

N_NODES = 10000
D_FEAT = 128
N_EDGES = 320000

NC = 2
NS = 16
N_TILES = NC * NS

EDGES_PER_TILE = N_EDGES // N_TILES
CHUNK = 80
ITERS = EDGES_PER_TILE // CHUNK
N_PAD = 10112
ROWS_PER_TILE = N_PAD // NS


def _sc_scatter_add(x, src3, dst3, zeros):
  mesh = plsc.VectorSubcoreMesh(core_axis_name="c", subcore_axis_name="s")

  @functools.partial(
      pl.kernel,
      mesh=mesh,
      out_type=jax.ShapeDtypeStruct((NC, N_PAD, D_FEAT), jnp.float32),
      scratch_types=[
          pltpu.VMEM((ITERS * CHUNK,), jnp.int32),
          pltpu.VMEM((ITERS, CHUNK), jnp.int32),
          pltpu.VMEM((CHUNK, D_FEAT), jnp.float32),
          pltpu.VMEM((CHUNK, D_FEAT), jnp.float32),
          pltpu.VMEM_SHARED((N_PAD, D_FEAT), jnp.float32),
          pltpu.SemaphoreType.DMA,
          pltpu.SemaphoreType.DMA,
      ],
  )
  def body(x_hbm, src_hbm, dst_hbm, zeros_hbm, out_hbm,
           sidx, didx, rows0, rows1, ysh, sem0, sem1):
    c = lax.axis_index("c")
    s = lax.axis_index("s")
    wid = c * NS + s
    row0 = pl.multiple_of(s * ROWS_PER_TILE, 8)
    pltpu.sync_copy(zeros_hbm.at[pl.ds(row0, ROWS_PER_TILE)],
                    ysh.at[pl.ds(row0, ROWS_PER_TILE)])
    pltpu.sync_copy(src_hbm.at[wid], sidx)
    pltpu.sync_copy(dst_hbm.at[wid], didx)

    def start_gather(i, rows, sem):
      pltpu.async_copy(x_hbm.at[sidx.at[pl.ds(i * CHUNK, CHUNK)]], rows, sem)

    def wait_gather(i, rows, sem):
      pltpu.make_async_copy(
          x_hbm.at[sidx.at[pl.ds(i * CHUNK, CHUNK)]], rows, sem).wait()

    start_gather(0, rows0, sem0)
    plsc.subcore_barrier()

    def pair(g, carry):
      i0 = 2 * g
      start_gather(i0 + 1, rows1, sem1)
      wait_gather(i0, rows0, sem0)
      pltpu.sync_copy(rows0, ysh.at[didx.at[i0]], add=True)
      start_gather(i0 + 2, rows0, sem0)
      wait_gather(i0 + 1, rows1, sem1)
      pltpu.sync_copy(rows1, ysh.at[didx.at[i0 + 1]], add=True)
      return carry

    lax.fori_loop(0, (ITERS - 1) // 2, pair, 0)
    wait_gather(ITERS - 1, rows0, sem0)
    pltpu.sync_copy(rows0, ysh.at[didx.at[ITERS - 1]], add=True)

    plsc.subcore_barrier()
    pltpu.sync_copy(ysh.at[pl.ds(row0, ROWS_PER_TILE)],
                    out_hbm.at[c].at[pl.ds(row0, ROWS_PER_TILE)])

  return body(x, src3, dst3, zeros)


def _tc_linear(y2, W, b):
  BM = 400

  def body(y_ref, w_ref, b_ref, o_ref):
    ysum = y_ref[0] + y_ref[1]
    o_ref[...] = lax.dot_general(
        ysum, w_ref[...], (((1,), (1,)), ((), ())),
        preferred_element_type=jnp.float32) + b_ref[...]

  return pl.pallas_call(
      body,
      grid=(N_NODES // BM,),
      in_specs=[
          pl.BlockSpec((NC, BM, D_FEAT), lambda i: (0, i, 0)),
          pl.BlockSpec((D_FEAT, D_FEAT), lambda i: (0, 0)),
          pl.BlockSpec((1, D_FEAT), lambda i: (0, 0)),
      ],
      out_specs=pl.BlockSpec((BM, D_FEAT), lambda i: (i, 0)),
      out_shape=jax.ShapeDtypeStruct((N_NODES, D_FEAT), jnp.float32),
  )(y2, W, b.reshape(1, D_FEAT))


def kernel(x, edge_index, W, b):
  src = edge_index[0].astype(jnp.int32).reshape(N_TILES, ITERS * CHUNK)
  dst = edge_index[1].astype(jnp.int32).reshape(N_TILES, ITERS, CHUNK)
  zeros = jnp.zeros((N_PAD, D_FEAT), jnp.float32)
  y2 = _sc_scatter_add(x, src, dst, zeros)
  return _tc_linear(y2, W, b)


# speedup vs baseline: 12.1254x; 1.0762x over previous
import functools

import jax
import jax.numpy as jnp
from jax import lax
from jax.experimental import pallas as pl
from jax.experimental.pallas import tpu as pltpu
from jax.experimental.pallas import tpu_sc as plsc

N_NODES = 10000
D_FEAT = 128
D_HALF = D_FEAT // 2
N_EDGES = 320000

NC = 2
NS = 16

EDGES_PER_TILE = N_EDGES // NS
CHUNK = 80
ITERS = EDGES_PER_TILE // CHUNK
RING = 5
GRP = ITERS // RING - 2
N_PAD = 10112
ROWS_PER_TILE = N_PAD // NS


def _sc_scatter_add(xstk, edge3, zeros):
  mesh = plsc.VectorSubcoreMesh(core_axis_name="c", subcore_axis_name="s")

  @functools.partial(
      pl.kernel,
      mesh=mesh,
      compiler_params=pltpu.CompilerParams(use_tc_tiling_on_sc=False),
      out_type=jax.ShapeDtypeStruct((NC, N_PAD, D_HALF), jnp.float32),
      scratch_types=[
          pltpu.VMEM((ITERS, CHUNK), jnp.int32),
          pltpu.VMEM((ITERS, CHUNK), jnp.int32),
          [pltpu.VMEM((CHUNK, D_HALF), jnp.float32) for _ in range(RING)],
          pltpu.VMEM_SHARED((N_PAD, D_HALF), jnp.float32),
          [pltpu.SemaphoreType.DMA for _ in range(RING)],
          [pltpu.SemaphoreType.DMA for _ in range(RING)],
      ],
  )
  def body(x_hbm, e_hbm, zeros_hbm, out_hbm, sidx, didx, rows, ysh, gsem, ssem):
    c = lax.axis_index("c")
    s = lax.axis_index("s")
    row0 = pl.multiple_of(s * ROWS_PER_TILE, 8)
    pltpu.sync_copy(zeros_hbm, ysh.at[pl.ds(row0, ROWS_PER_TILE)])
    pltpu.sync_copy(e_hbm.at[0].at[s], sidx)
    pltpu.sync_copy(e_hbm.at[1].at[s], didx)

    def fire_g(i, j):
      pltpu.async_copy(x_hbm.at[c].at[sidx.at[i]], rows[j], gsem[j])

    def wait_g(i, j):
      pltpu.make_async_copy(x_hbm.at[c].at[sidx.at[i]], rows[j], gsem[j]).wait()

    def fire_s(i, j):
      pltpu.async_copy(rows[j], ysh.at[didx.at[i]], ssem[j], add=True)

    def wait_s(i, j):
      pltpu.make_async_copy(rows[j], ysh.at[didx.at[i]], ssem[j]).wait()

    for j in range(3):
      fire_g(j, j)
    plsc.subcore_barrier()

    for j in range(RING):
      wait_g(j, j)
      fire_s(j, j)
      if j >= 2:
        wait_s(j - 2, (j + 3) % RING)
      fire_g(j + 3, (j + 3) % RING)

    def group(g, carry):
      i0 = 5 * (g + 1)
      for j in range(RING):
        i = i0 + j
        wait_g(i, j)
        fire_s(i, j)
        wait_s(i - 2, (j + 3) % RING)
        fire_g(i + 3, (j + 3) % RING)
      return carry

    lax.fori_loop(0, GRP, group, 0)

    i0 = ITERS - RING
    for j in range(RING):
      i = i0 + j
      wait_g(i, j)
      fire_s(i, j)
      wait_s(i - 2, (j + 3) % RING)
      if j < 2:
        fire_g(i + 3, (j + 3) % RING)
    wait_s(ITERS - 2, 3)
    wait_s(ITERS - 1, 4)

    plsc.subcore_barrier()
    pltpu.sync_copy(ysh.at[pl.ds(row0, ROWS_PER_TILE)],
                    out_hbm.at[c].at[pl.ds(row0, ROWS_PER_TILE)])

  return body(xstk, edge3, zeros)


def _tc_linear(y2, W, b):
  BM = 2000

  def body(y_ref, w_ref, b_ref, o_ref):
    dn = (((1,), (1,)), ((), ()))
    o_ref[...] = (
        lax.dot_general(y_ref[0], w_ref[:, :D_HALF], dn,
                        preferred_element_type=jnp.float32)
        + lax.dot_general(y_ref[1], w_ref[:, D_HALF:], dn,
                          preferred_element_type=jnp.float32)
        + b_ref[...])

  return pl.pallas_call(
      body,
      grid=(N_NODES // BM,),
      in_specs=[
          pl.BlockSpec((NC, BM, D_HALF), lambda i: (0, i, 0)),
          pl.BlockSpec((D_FEAT, D_FEAT), lambda i: (0, 0)),
          pl.BlockSpec((1, D_FEAT), lambda i: (0, 0)),
      ],
      out_specs=pl.BlockSpec((BM, D_FEAT), lambda i: (i, 0)),
      out_shape=jax.ShapeDtypeStruct((N_NODES, D_FEAT), jnp.float32),
  )(y2, W, b.reshape(1, D_FEAT))


def kernel(x, edge_index, W, b):
  if edge_index.dtype != jnp.int32:
    edge_index = edge_index.astype(jnp.int32)
  xstk = jnp.stack([x[:, :D_HALF], x[:, D_HALF:]])
  edge3 = edge_index.reshape(2, NS, ITERS, CHUNK)
  zeros = jnp.zeros((ROWS_PER_TILE, D_HALF), jnp.float32)
  y2 = _sc_scatter_add(xstk, edge3, zeros)
  return _tc_linear(y2, W, b)


# speedup vs baseline: 14.0738x; 1.1607x over previous
import functools

import jax
import jax.numpy as jnp
from jax import lax
from jax.experimental import pallas as pl
from jax.experimental.pallas import tpu as pltpu
from jax.experimental.pallas import tpu_sc as plsc

N_NODES = 10000
D_FEAT = 128
N_EDGES = 320000

NC = 2
NS = 16
N_TILES = NC * NS

EDGES_PER_TILE = N_EDGES // N_TILES
CHUNK = 40
ITERS = EDGES_PER_TILE // CHUNK
RING = 5
GRP = ITERS // RING - 2
N_PAD = 10112
ROWS_PER_TILE = N_PAD // NS


def _sc_scatter_add(x, edge3, zeros):
  mesh = plsc.VectorSubcoreMesh(core_axis_name="c", subcore_axis_name="s")

  @functools.partial(
      pl.kernel,
      mesh=mesh,
      compiler_params=pltpu.CompilerParams(use_tc_tiling_on_sc=False),
      out_type=jax.ShapeDtypeStruct((NC, N_PAD, D_FEAT), jnp.float32),
      scratch_types=[
          pltpu.VMEM((ITERS, CHUNK), jnp.int32),
          pltpu.VMEM((ITERS, CHUNK), jnp.int32),
          [pltpu.VMEM((CHUNK, D_FEAT), jnp.float32) for _ in range(RING)],
          pltpu.VMEM_SHARED((N_PAD, D_FEAT), jnp.float32),
          [pltpu.SemaphoreType.DMA for _ in range(RING)],
          [pltpu.SemaphoreType.DMA for _ in range(RING)],
      ],
  )
  def body(x_hbm, e_hbm, zeros_hbm, out_hbm, sidx, didx, rows, ysh, gsem, ssem):
    c = lax.axis_index("c")
    s = lax.axis_index("s")
    wid = c * NS + s
    row0 = pl.multiple_of(s * ROWS_PER_TILE, 8)
    pltpu.sync_copy(zeros_hbm, ysh.at[pl.ds(row0, ROWS_PER_TILE)])
    pltpu.sync_copy(e_hbm.at[0].at[wid], sidx)
    pltpu.sync_copy(e_hbm.at[1].at[wid], didx)

    def fire_g(i, j):
      pltpu.async_copy(x_hbm.at[sidx.at[i]], rows[j], gsem[j])

    def wait_g(i, j):
      pltpu.make_async_copy(x_hbm.at[sidx.at[i]], rows[j], gsem[j]).wait()

    def fire_s(i, j):
      pltpu.async_copy(rows[j], ysh.at[didx.at[i]], ssem[j], add=True)

    def wait_s(i, j):
      pltpu.make_async_copy(rows[j], ysh.at[didx.at[i]], ssem[j]).wait()

    for j in range(3):
      fire_g(j, j)
    plsc.subcore_barrier()

    for j in range(RING):
      wait_g(j, j)
      fire_s(j, j)
      if j >= 2:
        wait_s(j - 2, (j + 3) % RING)
      fire_g(j + 3, (j + 3) % RING)

    def group(g, carry):
      i0 = RING * (g + 1)
      for j in range(RING):
        i = i0 + j
        wait_g(i, j)
        fire_s(i, j)
        wait_s(i - 2, (j + 3) % RING)
        fire_g(i + 3, (j + 3) % RING)
      return carry

    lax.fori_loop(0, GRP, group, 0)

    i0 = ITERS - RING
    for j in range(RING):
      i = i0 + j
      wait_g(i, j)
      fire_s(i, j)
      wait_s(i - 2, (j + 3) % RING)
      if j < 2:
        fire_g(i + 3, (j + 3) % RING)
    wait_s(ITERS - 2, 3)
    wait_s(ITERS - 1, 4)

    plsc.subcore_barrier()
    pltpu.sync_copy(ysh.at[pl.ds(row0, ROWS_PER_TILE)],
                    out_hbm.at[c].at[pl.ds(row0, ROWS_PER_TILE)])

  return body(x, edge3, zeros)


def _tc_linear(y2, W, b):
  BM = 2000

  def body(y_ref, w_ref, b_ref, o_ref):
    ysum = y_ref[0] + y_ref[1]
    o_ref[...] = lax.dot_general(
        ysum, w_ref[...], (((1,), (1,)), ((), ())),
        preferred_element_type=jnp.float32) + b_ref[...]

  return pl.pallas_call(
      body,
      grid=(N_NODES // BM,),
      in_specs=[
          pl.BlockSpec((NC, BM, D_FEAT), lambda i: (0, i, 0)),
          pl.BlockSpec((D_FEAT, D_FEAT), lambda i: (0, 0)),
          pl.BlockSpec((1, D_FEAT), lambda i: (0, 0)),
      ],
      out_specs=pl.BlockSpec((BM, D_FEAT), lambda i: (i, 0)),
      out_shape=jax.ShapeDtypeStruct((N_NODES, D_FEAT), jnp.float32),
  )(y2, W, b.reshape(1, D_FEAT))


def kernel(x, edge_index, W, b):
  if edge_index.dtype != jnp.int32:
    edge_index = edge_index.astype(jnp.int32)
  edge3 = edge_index.reshape(2, N_TILES, ITERS, CHUNK)
  zeros = jnp.zeros((ROWS_PER_TILE, D_FEAT), jnp.float32)
  y2 = _sc_scatter_add(x, edge3, zeros)
  return _tc_linear(y2, W, b)


# speedup vs baseline: 15.0137x; 1.0668x over previous
import functools

import jax
import jax.numpy as jnp
from jax import lax
from jax.experimental import pallas as pl
from jax.experimental.pallas import tpu as pltpu
from jax.experimental.pallas import tpu_sc as plsc

N_NODES = 10000
D_FEAT = 128
N_EDGES = 320000

NC = 2
NS = 16
N_TILES = NC * NS

EDGES_PER_TILE = N_EDGES // N_TILES
CHUNK = 80
ITERS = EDGES_PER_TILE // CHUNK
RING = 3
GRP = (ITERS - 2) // RING - 1
N_PAD = N_NODES
ROWS_PER_TILE = N_PAD // NS


def _sc_scatter_add(x, edge3, zeros):
  mesh = plsc.VectorSubcoreMesh(core_axis_name="c", subcore_axis_name="s")

  @functools.partial(
      pl.kernel,
      mesh=mesh,
      compiler_params=pltpu.CompilerParams(use_tc_tiling_on_sc=False),
      out_type=jax.ShapeDtypeStruct((NC, N_PAD, D_FEAT), jnp.float32),
      scratch_types=[
          pltpu.VMEM((ITERS, CHUNK), jnp.int32),
          pltpu.VMEM((ITERS, CHUNK), jnp.int32),
          [pltpu.VMEM((CHUNK, D_FEAT), jnp.float32) for _ in range(RING)],
          pltpu.VMEM_SHARED((N_PAD, D_FEAT), jnp.float32),
          [pltpu.SemaphoreType.DMA for _ in range(RING)],
          [pltpu.SemaphoreType.DMA for _ in range(RING)],
      ],
  )
  def body(x_hbm, e_hbm, zeros_hbm, out_hbm, sidx, didx, rows, ysh, gsem, ssem):
    c = lax.axis_index("c")
    s = lax.axis_index("s")
    wid = c * NS + s
    row0 = pl.multiple_of(s * ROWS_PER_TILE, 8)
    pltpu.sync_copy(zeros_hbm, ysh.at[pl.ds(row0, ROWS_PER_TILE)])
    pltpu.sync_copy(e_hbm.at[0].at[wid], sidx)
    pltpu.sync_copy(e_hbm.at[1].at[wid], didx)

    def fire_g(i, j):
      pltpu.async_copy(x_hbm.at[sidx.at[i]], rows[j], gsem[j])

    def wait_g(i, j):
      pltpu.make_async_copy(x_hbm.at[sidx.at[i]], rows[j], gsem[j]).wait()

    def fire_s(i, j):
      pltpu.async_copy(rows[j], ysh.at[didx.at[i]], ssem[j], add=True)

    def wait_s(i, j):
      pltpu.make_async_copy(rows[j], ysh.at[didx.at[i]], ssem[j]).wait()

    fire_g(0, 0)
    fire_g(1, 1)
    plsc.subcore_barrier()

    wait_g(0, 0)
    fire_s(0, 0)
    fire_g(2, 2)
    wait_g(1, 1)
    fire_s(1, 1)
    wait_s(0, 0)
    fire_g(3, 0)

    def step(i, sj):
      wait_g(i, sj)
      fire_s(i, sj)
      wait_s(i - 1, (sj + 2) % RING)
      fire_g(i + 2, (sj + 2) % RING)

    def group(g, carry):
      i0 = 2 + RING * g
      for j in range(RING):
        step(i0 + j, (2 + j) % RING)
      return carry

    lax.fori_loop(0, GRP, group, 0)

    i0 = ITERS - RING
    wait_g(i0, i0 % RING)
    fire_s(i0, i0 % RING)
    wait_s(i0 - 1, (i0 + 2) % RING)
    fire_g(i0 + 2, (i0 + 2) % RING)
    wait_g(i0 + 1, (i0 + 1) % RING)
    fire_s(i0 + 1, (i0 + 1) % RING)
    wait_s(i0, i0 % RING)
    wait_g(i0 + 2, (i0 + 2) % RING)
    fire_s(i0 + 2, (i0 + 2) % RING)
    wait_s(i0 + 1, (i0 + 1) % RING)
    wait_s(i0 + 2, (i0 + 2) % RING)

    plsc.subcore_barrier()
    pltpu.sync_copy(ysh.at[pl.ds(row0, ROWS_PER_TILE)],
                    out_hbm.at[c].at[pl.ds(row0, ROWS_PER_TILE)])

  return body(x, edge3, zeros)


def _tc_linear(y2, W, b):
  BM = 2000

  def body(y_ref, w_ref, b_ref, o_ref):
    ysum = y_ref[0] + y_ref[1]
    o_ref[...] = lax.dot_general(
        ysum, w_ref[...], (((1,), (1,)), ((), ())),
        preferred_element_type=jnp.float32) + b_ref[...]

  return pl.pallas_call(
      body,
      grid=(N_NODES // BM,),
      in_specs=[
          pl.BlockSpec((NC, BM, D_FEAT), lambda i: (0, i, 0)),
          pl.BlockSpec((D_FEAT, D_FEAT), lambda i: (0, 0)),
          pl.BlockSpec((1, D_FEAT), lambda i: (0, 0)),
      ],
      out_specs=pl.BlockSpec((BM, D_FEAT), lambda i: (i, 0)),
      out_shape=jax.ShapeDtypeStruct((N_NODES, D_FEAT), jnp.float32),
  )(y2, W, b.reshape(1, D_FEAT))


def kernel(x, edge_index, W, b):
  if edge_index.dtype != jnp.int32:
    edge_index = edge_index.astype(jnp.int32)
  edge3 = edge_index.reshape(2, N_TILES, ITERS, CHUNK)
  zeros = jnp.zeros((ROWS_PER_TILE, D_FEAT), jnp.float32)
  y2 = _sc_scatter_add(x, edge3, zeros)
  return _tc_linear(y2, W, b)
